# Initial kernel scaffold; baseline (speedup 1.0000x reference)
import functools

import jax
import jax.numpy as jnp
from jax import lax
from jax.experimental import pallas as pl
from jax.experimental.pallas import tpu as pltpu
from jax.experimental.pallas import tpu_sc as plsc

N = 10000
E = 320000
D = 128
NC = 2
NS = 16
NW = NC * NS
CHUNK = 128
ROWS = 2560
EPAD = ROWS * CHUNK
RPW = ROWS // NW
ZROWS = 10240
TRASH = 10200
ZSTRIPE = ZROWS // NS
OSTRIPE = N // NS
DEGW = 16
RBLK = 1000
GRID = N // RBLK

_sc_mesh = plsc.VectorSubcoreMesh(
    core_axis_name="c", subcore_axis_name="s", num_cores=NC, num_subcores=NS
)


@functools.partial(
    pl.kernel,
    out_type=jax.ShapeDtypeStruct((NC, N, DEGW), jnp.float32),
    mesh=_sc_mesh,
    scratch_types=[
        pltpu.VMEM((1, CHUNK), jnp.int32),
        pltpu.VMEM((CHUNK, DEGW), jnp.float32),
        pltpu.VMEM((ZSTRIPE, DEGW), jnp.float32),
        pltpu.VMEM_SHARED((ZROWS, DEGW), jnp.float32),
    ],
)
def _deg_kernel(dst_hbm, ones_hbm, zer_hbm, out_hbm, didx, onesv, zbuf, acc):
    c = lax.axis_index("c")
    s = lax.axis_index("s")
    wid = c * NS + s
    pltpu.sync_copy(zer_hbm, zbuf)
    pltpu.sync_copy(zbuf, acc.at[pl.ds(s * ZSTRIPE, ZSTRIPE)])
    pltpu.sync_copy(ones_hbm, onesv)
    plsc.subcore_barrier()

    def body(j, carry):
        pltpu.sync_copy(dst_hbm.at[wid * RPW + j], didx.at[0])
        pltpu.sync_copy(onesv, acc.at[didx.at[0]], add=True)
        return carry

    lax.fori_loop(0, RPW, body, 0)
    plsc.subcore_barrier()
    pltpu.sync_copy(
        acc.at[pl.ds(s * OSTRIPE, OSTRIPE)],
        out_hbm.at[c, pl.ds(s * OSTRIPE, OSTRIPE)],
    )


@functools.partial(
    pl.kernel,
    out_type=jax.ShapeDtypeStruct((NC, N, D), jnp.float32),
    mesh=_sc_mesh,
    scratch_types=[
        pltpu.VMEM((2, CHUNK), jnp.int32),
        pltpu.VMEM((2, CHUNK), jnp.int32),
        pltpu.VMEM((2, CHUNK, D), jnp.float32),
        pltpu.VMEM((CHUNK, D), jnp.float32),
        pltpu.VMEM_SHARED((ZROWS, D), jnp.float32),
        pltpu.SemaphoreType.DMA,
    ],
)
def _edge_kernel(y_hbm, src_hbm, dst_hbm, zer_hbm, out_hbm,
                 sidx, didx, rows, zbuf, acc, sem):
    c = lax.axis_index("c")
    s = lax.axis_index("s")
    wid = c * NS + s
    pltpu.sync_copy(zer_hbm, zbuf)
    for i in range(ZSTRIPE // CHUNK):
        pltpu.sync_copy(zbuf, acc.at[pl.ds(s * ZSTRIPE + i * CHUNK, CHUNK)])
    plsc.subcore_barrier()

    def body(j, carry):
        row = wid * RPW + j
        pltpu.sync_copy(src_hbm.at[row], sidx.at[0])
        pltpu.sync_copy(dst_hbm.at[row], didx.at[0])
        pltpu.async_copy(y_hbm.at[sidx.at[0]], rows.at[0], sem).wait()
        pltpu.sync_copy(rows.at[0], acc.at[didx.at[0]], add=True)
        return carry

    lax.fori_loop(0, RPW, body, 0)
    plsc.subcore_barrier()
    pltpu.sync_copy(
        acc.at[pl.ds(s * OSTRIPE, OSTRIPE)],
        out_hbm.at[c, pl.ds(s * OSTRIPE, OSTRIPE)],
    )


def _dinv_body(t_ref, o_ref):
    t = t_ref[...]
    d = t[:, 0:1] + t[:, 1:2] + 1.0
    o_ref[...] = jnp.broadcast_to(lax.rsqrt(d), (RBLK, D))


_dinv_call = pl.pallas_call(
    _dinv_body,
    grid=(GRID,),
    in_specs=[pl.BlockSpec((RBLK, 2), lambda i: (i, 0))],
    out_specs=pl.BlockSpec((RBLK, D), lambda i: (i, 0)),
    out_shape=jax.ShapeDtypeStruct((N, D), jnp.float32),
)


def _mm_scale_body(x_ref, w_ref, dinv_ref, o_ref):
    xw = jnp.dot(x_ref[...], w_ref[...], preferred_element_type=jnp.float32)
    o_ref[...] = xw * dinv_ref[...]


_mm_scale = pl.pallas_call(
    _mm_scale_body,
    grid=(GRID,),
    in_specs=[
        pl.BlockSpec((RBLK, D), lambda i: (i, 0)),
        pl.BlockSpec((D, D), lambda i: (0, 0)),
        pl.BlockSpec((RBLK, D), lambda i: (i, 0)),
    ],
    out_specs=pl.BlockSpec((RBLK, D), lambda i: (i, 0)),
    out_shape=jax.ShapeDtypeStruct((N, D), jnp.float32),
)


def _layer2_body(z_ref, y_ref, dinv_ref, b_ref, w_ref, o_ref):
    zsum = z_ref[0] + z_ref[1]
    h = jnp.maximum(dinv_ref[...] * (zsum + y_ref[...]) + b_ref[...], 0.0)
    hw = jnp.dot(h, w_ref[...], preferred_element_type=jnp.float32)
    o_ref[...] = hw * dinv_ref[...]


_layer2 = pl.pallas_call(
    _layer2_body,
    grid=(GRID,),
    in_specs=[
        pl.BlockSpec((NC, RBLK, D), lambda i: (0, i, 0)),
        pl.BlockSpec((RBLK, D), lambda i: (i, 0)),
        pl.BlockSpec((RBLK, D), lambda i: (i, 0)),
        pl.BlockSpec((D,), lambda i: (0,)),
        pl.BlockSpec((D, D), lambda i: (0, 0)),
    ],
    out_specs=pl.BlockSpec((RBLK, D), lambda i: (i, 0)),
    out_shape=jax.ShapeDtypeStruct((N, D), jnp.float32),
)


def _final_body(z_ref, y_ref, dinv_ref, b_ref, o_ref):
    o = dinv_ref[...] * (z_ref[0] + z_ref[1] + y_ref[...]) + b_ref[...]
    m = jnp.max(o, axis=1, keepdims=True)
    t = o - m
    o_ref[...] = t - jnp.log(jnp.sum(jnp.exp(t), axis=1, keepdims=True))


_final = pl.pallas_call(
    _final_body,
    grid=(GRID,),
    in_specs=[
        pl.BlockSpec((NC, RBLK, D), lambda i: (0, i, 0)),
        pl.BlockSpec((RBLK, D), lambda i: (i, 0)),
        pl.BlockSpec((RBLK, D), lambda i: (i, 0)),
        pl.BlockSpec((D,), lambda i: (0,)),
    ],
    out_specs=pl.BlockSpec((RBLK, D), lambda i: (i, 0)),
    out_shape=jax.ShapeDtypeStruct((N, D), jnp.float32),
)


def kernel(x, edge_index, W1, b1, W2, b2):
    src = edge_index[0].astype(jnp.int32)
    dst = edge_index[1].astype(jnp.int32)
    pad = EPAD - E
    srcp = jnp.concatenate([src, jnp.zeros((pad,), jnp.int32)]).reshape(ROWS, CHUNK)
    dstp = jnp.concatenate([dst, jnp.full((pad,), TRASH, jnp.int32)]).reshape(ROWS, CHUNK)
    ones_rows = jnp.ones((CHUNK, DEGW), jnp.float32)
    zer_deg = jnp.zeros((ZSTRIPE, DEGW), jnp.float32)
    zer_d = jnp.zeros((CHUNK, D), jnp.float32)

    degp = _deg_kernel(dstp, ones_rows, zer_deg)
    t = jnp.transpose(degp[:, :, 0])
    dinvb = _dinv_call(t)
    y1 = _mm_scale(x, W1, dinvb)
    z1 = _edge_kernel(y1, srcp, dstp, zer_d)
    y2 = _layer2(z1, y1, dinvb, b1, W2)
    z2 = _edge_kernel(y2, srcp, dstp, zer_d)
    return _final(z2, y2, dinvb, b2)


# speedup vs baseline: 7.2844x; 7.2844x over previous
import functools

import jax
import jax.numpy as jnp
from jax import lax
from jax.experimental import pallas as pl
from jax.experimental.pallas import tpu as pltpu
from jax.experimental.pallas import tpu_sc as plsc

N = 10000
E = 320000
D = 128
NC = 2
NS = 16
NW = NC * NS
CHUNK = 128
ROWS = 2560
EPAD = ROWS * CHUNK
RPW = ROWS // NW
ZROWS = 10112
TRASH = 10100
ZSTRIPE = ZROWS // NS
DEGW = 128
RBLK = 1000
GRID = N // RBLK

_sc_mesh = plsc.VectorSubcoreMesh(
    core_axis_name="c", subcore_axis_name="s", num_cores=NC, num_subcores=NS
)


@functools.partial(
    pl.kernel,
    out_type=jax.ShapeDtypeStruct((NC, ZROWS, DEGW), jnp.float32),
    mesh=_sc_mesh,
    scratch_types=[
        pltpu.VMEM((1, CHUNK), jnp.int32),
        pltpu.VMEM((CHUNK, DEGW), jnp.float32),
        pltpu.VMEM((CHUNK, DEGW), jnp.float32),
        pltpu.VMEM_SHARED((ZROWS, DEGW), jnp.float32),
    ],
)
def _deg_kernel(dst_hbm, ones_hbm, zer_hbm, out_hbm, didx, onesv, zbuf, acc):
    c = lax.axis_index("c")
    s = lax.axis_index("s")
    wid = c * NS + s
    pltpu.sync_copy(zer_hbm, zbuf)
    for i in range(ZSTRIPE // CHUNK):
        pltpu.sync_copy(zbuf, acc.at[pl.ds(s * ZSTRIPE + i * CHUNK, CHUNK)])
    rem = ZSTRIPE % CHUNK
    if rem:
        pltpu.sync_copy(
            zbuf.at[pl.ds(0, rem)],
            acc.at[pl.ds(s * ZSTRIPE + ZSTRIPE - rem, rem)],
        )
    pltpu.sync_copy(ones_hbm, onesv)
    plsc.subcore_barrier()

    def body(j, carry):
        pltpu.sync_copy(dst_hbm.at[wid * RPW + j], didx.at[0])
        pltpu.sync_copy(onesv, acc.at[didx.at[0]], add=True)
        return carry

    lax.fori_loop(0, RPW, body, 0)
    plsc.subcore_barrier()
    pltpu.sync_copy(
        acc.at[pl.ds(s * ZSTRIPE, ZSTRIPE)],
        out_hbm.at[c, pl.ds(s * ZSTRIPE, ZSTRIPE)],
    )


@functools.partial(
    pl.kernel,
    out_type=jax.ShapeDtypeStruct((NC, ZROWS, D), jnp.float32),
    mesh=_sc_mesh,
    scratch_types=[
        pltpu.VMEM((2, CHUNK), jnp.int32),
        pltpu.VMEM((2, CHUNK), jnp.int32),
        pltpu.VMEM((2, CHUNK, D), jnp.float32),
        pltpu.VMEM((CHUNK, D), jnp.float32),
        pltpu.VMEM_SHARED((ZROWS, D), jnp.float32),
        pltpu.SemaphoreType.DMA,
    ],
)
def _edge_kernel(y_hbm, src_hbm, dst_hbm, zer_hbm, out_hbm,
                 sidx, didx, rows, zbuf, acc, sem):
    c = lax.axis_index("c")
    s = lax.axis_index("s")
    wid = c * NS + s
    pltpu.sync_copy(zer_hbm, zbuf)
    for i in range(ZSTRIPE // CHUNK):
        pltpu.sync_copy(zbuf, acc.at[pl.ds(s * ZSTRIPE + i * CHUNK, CHUNK)])
    rem = ZSTRIPE % CHUNK
    if rem:
        pltpu.sync_copy(
            zbuf.at[pl.ds(0, rem)],
            acc.at[pl.ds(s * ZSTRIPE + ZSTRIPE - rem, rem)],
        )
    plsc.subcore_barrier()

    def body(j, carry):
        row = wid * RPW + j
        pltpu.sync_copy(src_hbm.at[row], sidx.at[0])
        pltpu.sync_copy(dst_hbm.at[row], didx.at[0])
        pltpu.async_copy(y_hbm.at[sidx.at[0]], rows.at[0], sem).wait()
        pltpu.sync_copy(rows.at[0], acc.at[didx.at[0]], add=True)
        return carry

    lax.fori_loop(0, RPW, body, 0)
    plsc.subcore_barrier()
    pltpu.sync_copy(
        acc.at[pl.ds(s * ZSTRIPE, ZSTRIPE)],
        out_hbm.at[c, pl.ds(s * ZSTRIPE, ZSTRIPE)],
    )


def _dinv_body(t_ref, o_ref):
    t = t_ref[...]
    d = t[:, 0:1] + t[:, 1:2] + 1.0
    o_ref[...] = jnp.broadcast_to(lax.rsqrt(d), (RBLK, D))


_dinv_call = pl.pallas_call(
    _dinv_body,
    grid=(GRID,),
    in_specs=[pl.BlockSpec((RBLK, 2), lambda i: (i, 0))],
    out_specs=pl.BlockSpec((RBLK, D), lambda i: (i, 0)),
    out_shape=jax.ShapeDtypeStruct((N, D), jnp.float32),
)


def _mm_scale_body(x_ref, w_ref, dinv_ref, o_ref):
    xw = jnp.dot(x_ref[...], w_ref[...], preferred_element_type=jnp.float32)
    o_ref[...] = xw * dinv_ref[...]


_mm_scale = pl.pallas_call(
    _mm_scale_body,
    grid=(GRID,),
    in_specs=[
        pl.BlockSpec((RBLK, D), lambda i: (i, 0)),
        pl.BlockSpec((D, D), lambda i: (0, 0)),
        pl.BlockSpec((RBLK, D), lambda i: (i, 0)),
    ],
    out_specs=pl.BlockSpec((RBLK, D), lambda i: (i, 0)),
    out_shape=jax.ShapeDtypeStruct((N, D), jnp.float32),
)


def _layer2_body(z_ref, y_ref, dinv_ref, b_ref, w_ref, o_ref):
    zsum = z_ref[0] + z_ref[1]
    h = jnp.maximum(dinv_ref[...] * (zsum + y_ref[...]) + b_ref[...], 0.0)
    hw = jnp.dot(h, w_ref[...], preferred_element_type=jnp.float32)
    o_ref[...] = hw * dinv_ref[...]


_layer2 = pl.pallas_call(
    _layer2_body,
    grid=(GRID,),
    in_specs=[
        pl.BlockSpec((NC, RBLK, D), lambda i: (0, i, 0)),
        pl.BlockSpec((RBLK, D), lambda i: (i, 0)),
        pl.BlockSpec((RBLK, D), lambda i: (i, 0)),
        pl.BlockSpec((D,), lambda i: (0,)),
        pl.BlockSpec((D, D), lambda i: (0, 0)),
    ],
    out_specs=pl.BlockSpec((RBLK, D), lambda i: (i, 0)),
    out_shape=jax.ShapeDtypeStruct((N, D), jnp.float32),
)


def _final_body(z_ref, y_ref, dinv_ref, b_ref, o_ref):
    o = dinv_ref[...] * (z_ref[0] + z_ref[1] + y_ref[...]) + b_ref[...]
    m = jnp.max(o, axis=1, keepdims=True)
    t = o - m
    o_ref[...] = t - jnp.log(jnp.sum(jnp.exp(t), axis=1, keepdims=True))


_final = pl.pallas_call(
    _final_body,
    grid=(GRID,),
    in_specs=[
        pl.BlockSpec((NC, RBLK, D), lambda i: (0, i, 0)),
        pl.BlockSpec((RBLK, D), lambda i: (i, 0)),
        pl.BlockSpec((RBLK, D), lambda i: (i, 0)),
        pl.BlockSpec((D,), lambda i: (0,)),
    ],
    out_specs=pl.BlockSpec((RBLK, D), lambda i: (i, 0)),
    out_shape=jax.ShapeDtypeStruct((N, D), jnp.float32),
)


def kernel(x, edge_index, W1, b1, W2, b2):
    src = edge_index[0].astype(jnp.int32)
    dst = edge_index[1].astype(jnp.int32)
    pad = EPAD - E
    srcp = jnp.concatenate([src, jnp.zeros((pad,), jnp.int32)]).reshape(ROWS, CHUNK)
    dstp = jnp.concatenate([dst, jnp.full((pad,), TRASH, jnp.int32)]).reshape(ROWS, CHUNK)
    ones_rows = jnp.ones((CHUNK, DEGW), jnp.float32)
    zer_deg = jnp.zeros((CHUNK, DEGW), jnp.float32)
    zer_d = jnp.zeros((CHUNK, D), jnp.float32)

    degp = _deg_kernel(dstp, ones_rows, zer_deg)
    t = jnp.transpose(degp[:, :N, 0])
    dinvb = _dinv_call(t)
    y1 = _mm_scale(x, W1, dinvb)
    z1 = _edge_kernel(y1, srcp, dstp, zer_d)
    y2 = _layer2(z1, y1, dinvb, b1, W2)
    z2 = _edge_kernel(y2, srcp, dstp, zer_d)
    return _final(z2, y2, dinvb, b2)


# speedup vs baseline: 17.6359x; 2.4211x over previous
import functools

import jax
import jax.numpy as jnp
from jax import lax
from jax.experimental import pallas as pl
from jax.experimental.pallas import tpu as pltpu
from jax.experimental.pallas import tpu_sc as plsc

N = 10000
E = 320000
D = 128
NC = 2
NS = 16
NW = NC * NS
CHUNK = 128
ROWS = E // CHUNK
PAIRS = ROWS // 2
PPW = PAIRS // NW
ZROWS = 10112
ZSTRIPE = ZROWS // NS
DEGW = 128
RBLK = 1000
GRID = N // RBLK

_sc_mesh = plsc.VectorSubcoreMesh(
    core_axis_name="c", subcore_axis_name="s", num_cores=NC, num_subcores=NS
)


def _init_stripe(zer_hbm, zbuf, acc, s):
    pltpu.sync_copy(zer_hbm, zbuf)
    for i in range(ZSTRIPE // CHUNK):
        pltpu.sync_copy(zbuf, acc.at[pl.ds(s * ZSTRIPE + i * CHUNK, CHUNK)])
    rem = ZSTRIPE % CHUNK
    if rem:
        pltpu.sync_copy(
            zbuf.at[pl.ds(0, rem)],
            acc.at[pl.ds(s * ZSTRIPE + ZSTRIPE - rem, rem)],
        )


@functools.partial(
    pl.kernel,
    out_type=jax.ShapeDtypeStruct((NC, ZROWS, DEGW), jnp.float32),
    mesh=_sc_mesh,
    scratch_types=[
        pltpu.VMEM((2, 2, CHUNK), jnp.int32),
        pltpu.VMEM((CHUNK, DEGW), jnp.float32),
        pltpu.VMEM((CHUNK, DEGW), jnp.float32),
        pltpu.VMEM_SHARED((ZROWS, DEGW), jnp.float32),
    ],
)
def _deg_kernel(idx_hbm, ones_hbm, zer_hbm, out_hbm, idxq, onesv, zbuf, acc):
    c = lax.axis_index("c")
    s = lax.axis_index("s")
    wid = c * NS + s
    _init_stripe(zer_hbm, zbuf, acc, s)
    pltpu.sync_copy(ones_hbm, onesv)
    plsc.subcore_barrier()

    def pairstep(p):
        pltpu.sync_copy(idx_hbm.at[p], idxq)
        pltpu.sync_copy(onesv, acc.at[idxq.at[0, 1]], add=True)
        pltpu.sync_copy(onesv, acc.at[idxq.at[1, 1]], add=True)

    def body(j, carry):
        pairstep(wid * PPW + j)
        return carry

    lax.fori_loop(0, PPW, body, 0)

    @pl.when(wid < 2)
    def _():
        pairstep(NW * PPW + wid)

    plsc.subcore_barrier()
    pltpu.sync_copy(
        acc.at[pl.ds(s * ZSTRIPE, ZSTRIPE)],
        out_hbm.at[c, pl.ds(s * ZSTRIPE, ZSTRIPE)],
    )


@functools.partial(
    pl.kernel,
    out_type=jax.ShapeDtypeStruct((NC, ZROWS, D), jnp.float32),
    mesh=_sc_mesh,
    scratch_types=[
        pltpu.VMEM((2, 2, CHUNK), jnp.int32),
        pltpu.VMEM((CHUNK, D), jnp.float32),
        pltpu.VMEM((CHUNK, D), jnp.float32),
        pltpu.VMEM((CHUNK, D), jnp.float32),
        pltpu.VMEM_SHARED((ZROWS, D), jnp.float32),
        pltpu.SemaphoreType.DMA,
        pltpu.SemaphoreType.DMA,
    ],
)
def _edge_kernel(y_hbm, idx_hbm, zer_hbm, out_hbm,
                 idxq, rows_a, rows_b, zbuf, acc, sem_a, sem_b):
    c = lax.axis_index("c")
    s = lax.axis_index("s")
    wid = c * NS + s
    _init_stripe(zer_hbm, zbuf, acc, s)
    plsc.subcore_barrier()

    def pairstep(p):
        pltpu.sync_copy(idx_hbm.at[p], idxq)
        cp_a = pltpu.async_copy(y_hbm.at[idxq.at[0, 0]], rows_a, sem_a)
        cp_b = pltpu.async_copy(y_hbm.at[idxq.at[1, 0]], rows_b, sem_b)
        cp_a.wait()
        pltpu.sync_copy(rows_a, acc.at[idxq.at[0, 1]], add=True)
        cp_b.wait()
        pltpu.sync_copy(rows_b, acc.at[idxq.at[1, 1]], add=True)

    def body(j, carry):
        pairstep(wid * PPW + j)
        return carry

    lax.fori_loop(0, PPW, body, 0)

    @pl.when(wid < 2)
    def _():
        pairstep(NW * PPW + wid)

    plsc.subcore_barrier()
    pltpu.sync_copy(
        acc.at[pl.ds(s * ZSTRIPE, ZSTRIPE)],
        out_hbm.at[c, pl.ds(s * ZSTRIPE, ZSTRIPE)],
    )


def _dinv_body(t_ref, o_ref):
    t = t_ref[...]
    d = t[:, 0:1] + t[:, 1:2] + 1.0
    o_ref[...] = jnp.broadcast_to(lax.rsqrt(d), (RBLK, D))


_dinv_call = pl.pallas_call(
    _dinv_body,
    grid=(GRID,),
    in_specs=[pl.BlockSpec((RBLK, 2), lambda i: (i, 0))],
    out_specs=pl.BlockSpec((RBLK, D), lambda i: (i, 0)),
    out_shape=jax.ShapeDtypeStruct((N, D), jnp.float32),
)


def _mm_scale_body(x_ref, w_ref, dinv_ref, o_ref):
    xw = jnp.dot(x_ref[...], w_ref[...], preferred_element_type=jnp.float32)
    o_ref[...] = xw * dinv_ref[...]


_mm_scale = pl.pallas_call(
    _mm_scale_body,
    grid=(GRID,),
    in_specs=[
        pl.BlockSpec((RBLK, D), lambda i: (i, 0)),
        pl.BlockSpec((D, D), lambda i: (0, 0)),
        pl.BlockSpec((RBLK, D), lambda i: (i, 0)),
    ],
    out_specs=pl.BlockSpec((RBLK, D), lambda i: (i, 0)),
    out_shape=jax.ShapeDtypeStruct((N, D), jnp.float32),
)


def _layer2_body(z_ref, y_ref, dinv_ref, b_ref, w_ref, o_ref):
    zsum = z_ref[0] + z_ref[1]
    h = jnp.maximum(dinv_ref[...] * (zsum + y_ref[...]) + b_ref[...], 0.0)
    hw = jnp.dot(h, w_ref[...], preferred_element_type=jnp.float32)
    o_ref[...] = hw * dinv_ref[...]


_layer2 = pl.pallas_call(
    _layer2_body,
    grid=(GRID,),
    in_specs=[
        pl.BlockSpec((NC, RBLK, D), lambda i: (0, i, 0)),
        pl.BlockSpec((RBLK, D), lambda i: (i, 0)),
        pl.BlockSpec((RBLK, D), lambda i: (i, 0)),
        pl.BlockSpec((D,), lambda i: (0,)),
        pl.BlockSpec((D, D), lambda i: (0, 0)),
    ],
    out_specs=pl.BlockSpec((RBLK, D), lambda i: (i, 0)),
    out_shape=jax.ShapeDtypeStruct((N, D), jnp.float32),
)


def _final_body(z_ref, y_ref, dinv_ref, b_ref, o_ref):
    o = dinv_ref[...] * (z_ref[0] + z_ref[1] + y_ref[...]) + b_ref[...]
    m = jnp.max(o, axis=1, keepdims=True)
    t = o - m
    o_ref[...] = t - jnp.log(jnp.sum(jnp.exp(t), axis=1, keepdims=True))


_final = pl.pallas_call(
    _final_body,
    grid=(GRID,),
    in_specs=[
        pl.BlockSpec((NC, RBLK, D), lambda i: (0, i, 0)),
        pl.BlockSpec((RBLK, D), lambda i: (i, 0)),
        pl.BlockSpec((RBLK, D), lambda i: (i, 0)),
        pl.BlockSpec((D,), lambda i: (0,)),
    ],
    out_specs=pl.BlockSpec((RBLK, D), lambda i: (i, 0)),
    out_shape=jax.ShapeDtypeStruct((N, D), jnp.float32),
)


def kernel(x, edge_index, W1, b1, W2, b2):
    src = edge_index[0].astype(jnp.int32).reshape(PAIRS, 2, 1, CHUNK)
    dst = edge_index[1].astype(jnp.int32).reshape(PAIRS, 2, 1, CHUNK)
    idxp = jnp.concatenate([src, dst], axis=2)
    ones_rows = jnp.ones((CHUNK, DEGW), jnp.float32)
    zer_d = jnp.zeros((CHUNK, D), jnp.float32)

    degp = _deg_kernel(idxp, ones_rows, zer_d)
    t = jnp.transpose(degp[:, :N, 0])
    dinvb = _dinv_call(t)
    y1 = _mm_scale(x, W1, dinvb)
    z1 = _edge_kernel(y1, idxp, zer_d)
    y2 = _layer2(z1, y1, dinvb, b1, W2)
    z2 = _edge_kernel(y2, idxp, zer_d)
    return _final(z2, y2, dinvb, b2)


# speedup vs baseline: 20.2319x; 1.1472x over previous
import functools

import jax
import jax.numpy as jnp
from jax import lax
from jax.experimental import pallas as pl
from jax.experimental.pallas import tpu as pltpu
from jax.experimental.pallas import tpu_sc as plsc

N = 10000
E = 320000
D = 128
NC = 2
NS = 16
NW = NC * NS
CHUNK = 128
ROWS = E // CHUNK
PAIRS = ROWS // 2
PPW = PAIRS // NW
ZROWS = 10112
ZSTRIPE = ZROWS // NS
DEGW = 128
RBLK = 1000
GRID = N // RBLK

_sc_mesh = plsc.VectorSubcoreMesh(
    core_axis_name="c", subcore_axis_name="s", num_cores=NC, num_subcores=NS
)


def _init_stripe(zer_hbm, zbuf, acc, s):
    pltpu.sync_copy(zer_hbm, zbuf)
    for i in range(ZSTRIPE // CHUNK):
        pltpu.sync_copy(zbuf, acc.at[pl.ds(s * ZSTRIPE + i * CHUNK, CHUNK)])
    rem = ZSTRIPE % CHUNK
    if rem:
        pltpu.sync_copy(
            zbuf.at[pl.ds(0, rem)],
            acc.at[pl.ds(s * ZSTRIPE + ZSTRIPE - rem, rem)],
        )


@functools.partial(
    pl.kernel,
    out_type=jax.ShapeDtypeStruct((NC, ZROWS, DEGW), jnp.float32),
    mesh=_sc_mesh,
    scratch_types=[
        pltpu.VMEM((2, 2, CHUNK), jnp.int32),
        pltpu.VMEM((CHUNK, DEGW), jnp.float32),
        pltpu.VMEM((CHUNK, DEGW), jnp.float32),
        pltpu.VMEM_SHARED((ZROWS, DEGW), jnp.float32),
    ],
)
def _deg_kernel(idx_hbm, ones_hbm, zer_hbm, out_hbm, idxq, onesv, zbuf, acc):
    c = lax.axis_index("c")
    s = lax.axis_index("s")
    wid = c * NS + s
    _init_stripe(zer_hbm, zbuf, acc, s)
    pltpu.sync_copy(ones_hbm, onesv)
    plsc.subcore_barrier()

    def pairstep(p):
        pltpu.sync_copy(idx_hbm.at[p], idxq)
        pltpu.sync_copy(onesv, acc.at[idxq.at[0, 1]], add=True)
        pltpu.sync_copy(onesv, acc.at[idxq.at[1, 1]], add=True)

    def body(j, carry):
        pairstep(wid * PPW + j)
        return carry

    lax.fori_loop(0, PPW, body, 0)

    @pl.when(wid < 2)
    def _():
        pairstep(NW * PPW + wid)

    plsc.subcore_barrier()
    pltpu.sync_copy(
        acc.at[pl.ds(s * ZSTRIPE, ZSTRIPE)],
        out_hbm.at[c, pl.ds(s * ZSTRIPE, ZSTRIPE)],
    )


@functools.partial(
    pl.kernel,
    out_type=jax.ShapeDtypeStruct((NC, ZROWS, D), jnp.float32),
    mesh=_sc_mesh,
    scratch_types=[
        pltpu.VMEM((2, 2, CHUNK), jnp.int32),
        pltpu.VMEM((CHUNK, D), jnp.float32),
        pltpu.VMEM((CHUNK, D), jnp.float32),
        pltpu.VMEM((CHUNK, D), jnp.float32),
        pltpu.VMEM_SHARED((ZROWS, D), jnp.float32),
        pltpu.SemaphoreType.DMA,
        pltpu.SemaphoreType.DMA,
    ],
)
def _edge_kernel(y_hbm, idx_hbm, zer_hbm, out_hbm,
                 idxq, rows_a, rows_b, zbuf, acc, sem_a, sem_b):
    c = lax.axis_index("c")
    s = lax.axis_index("s")
    wid = c * NS + s
    _init_stripe(zer_hbm, zbuf, acc, s)
    plsc.subcore_barrier()

    def pairstep(p):
        pltpu.sync_copy(idx_hbm.at[p], idxq)
        cp_a = pltpu.async_copy(y_hbm.at[idxq.at[0, 0]], rows_a, sem_a)
        cp_b = pltpu.async_copy(y_hbm.at[idxq.at[1, 0]], rows_b, sem_b)
        cp_a.wait()
        pltpu.sync_copy(rows_a, acc.at[idxq.at[0, 1]], add=True)
        cp_b.wait()
        pltpu.sync_copy(rows_b, acc.at[idxq.at[1, 1]], add=True)

    def body(j, carry):
        pairstep(wid * PPW + j)
        return carry

    lax.fori_loop(0, PPW, body, 0)

    @pl.when(wid < 2)
    def _():
        pairstep(NW * PPW + wid)

    plsc.subcore_barrier()
    pltpu.sync_copy(
        acc.at[pl.ds(s * ZSTRIPE, ZSTRIPE)],
        out_hbm.at[c, pl.ds(s * ZSTRIPE, ZSTRIPE)],
    )


def _dinv_mm_body(dg_ref, x_ref, w_ref, y_ref, dinv_ref):
    dg = dg_ref[...]
    d = dg[0, :, 0:1] + dg[1, :, 0:1] + 1.0
    dinvb = jnp.broadcast_to(lax.rsqrt(d), (RBLK, D))
    xw = jnp.dot(x_ref[...], w_ref[...], preferred_element_type=jnp.float32)
    y_ref[...] = xw * dinvb
    dinv_ref[...] = dinvb


_dinv_mm = pl.pallas_call(
    _dinv_mm_body,
    grid=(GRID,),
    in_specs=[
        pl.BlockSpec((NC, RBLK, DEGW), lambda i: (0, i, 0)),
        pl.BlockSpec((RBLK, D), lambda i: (i, 0)),
        pl.BlockSpec((D, D), lambda i: (0, 0)),
    ],
    out_specs=[
        pl.BlockSpec((RBLK, D), lambda i: (i, 0)),
        pl.BlockSpec((RBLK, D), lambda i: (i, 0)),
    ],
    out_shape=[
        jax.ShapeDtypeStruct((N, D), jnp.float32),
        jax.ShapeDtypeStruct((N, D), jnp.float32),
    ],
)


def _layer2_body(z_ref, y_ref, dinv_ref, b_ref, w_ref, o_ref):
    zsum = z_ref[0] + z_ref[1]
    h = jnp.maximum(dinv_ref[...] * (zsum + y_ref[...]) + b_ref[...], 0.0)
    hw = jnp.dot(h, w_ref[...], preferred_element_type=jnp.float32)
    o_ref[...] = hw * dinv_ref[...]


_layer2 = pl.pallas_call(
    _layer2_body,
    grid=(GRID,),
    in_specs=[
        pl.BlockSpec((NC, RBLK, D), lambda i: (0, i, 0)),
        pl.BlockSpec((RBLK, D), lambda i: (i, 0)),
        pl.BlockSpec((RBLK, D), lambda i: (i, 0)),
        pl.BlockSpec((D,), lambda i: (0,)),
        pl.BlockSpec((D, D), lambda i: (0, 0)),
    ],
    out_specs=pl.BlockSpec((RBLK, D), lambda i: (i, 0)),
    out_shape=jax.ShapeDtypeStruct((N, D), jnp.float32),
)


def _final_body(z_ref, y_ref, dinv_ref, b_ref, o_ref):
    o = dinv_ref[...] * (z_ref[0] + z_ref[1] + y_ref[...]) + b_ref[...]
    m = jnp.max(o, axis=1, keepdims=True)
    t = o - m
    o_ref[...] = t - jnp.log(jnp.sum(jnp.exp(t), axis=1, keepdims=True))


_final = pl.pallas_call(
    _final_body,
    grid=(GRID,),
    in_specs=[
        pl.BlockSpec((NC, RBLK, D), lambda i: (0, i, 0)),
        pl.BlockSpec((RBLK, D), lambda i: (i, 0)),
        pl.BlockSpec((RBLK, D), lambda i: (i, 0)),
        pl.BlockSpec((D,), lambda i: (0,)),
    ],
    out_specs=pl.BlockSpec((RBLK, D), lambda i: (i, 0)),
    out_shape=jax.ShapeDtypeStruct((N, D), jnp.float32),
)


def kernel(x, edge_index, W1, b1, W2, b2):
    src = edge_index[0].astype(jnp.int32).reshape(PAIRS, 2, 1, CHUNK)
    dst = edge_index[1].astype(jnp.int32).reshape(PAIRS, 2, 1, CHUNK)
    idxp = jnp.concatenate([src, dst], axis=2)
    ones_rows = jnp.ones((CHUNK, DEGW), jnp.float32)
    zer_d = jnp.zeros((CHUNK, D), jnp.float32)

    degp = _deg_kernel(idxp, ones_rows, zer_d)
    y1, dinvb = _dinv_mm(degp, x, W1)
    z1 = _edge_kernel(y1, idxp, zer_d)
    y2 = _layer2(z1, y1, dinvb, b1, W2)
    z2 = _edge_kernel(y2, idxp, zer_d)
    return _final(z2, y2, dinvb, b2)


# speedup vs baseline: 26.9848x; 1.3338x over previous
import functools

import jax
import jax.numpy as jnp
from jax import lax
from jax.experimental import pallas as pl
from jax.experimental.pallas import tpu as pltpu
from jax.experimental.pallas import tpu_sc as plsc

N = 10000
E = 320000
D = 128
NC = 2
NS = 16
NW = NC * NS
CHUNK = 128
ROWS = E // CHUNK
PAIRS = ROWS // 2
PPW = PAIRS // NW
ZROWS = 10112
ZSTRIPE = ZROWS // NS
DEGW = 128
RBLK = 1000
GRID = N // RBLK

_sc_mesh = plsc.VectorSubcoreMesh(
    core_axis_name="c", subcore_axis_name="s", num_cores=NC, num_subcores=NS
)


def _init_stripe(zer_hbm, zbuf, acc, s):
    pltpu.sync_copy(zer_hbm, zbuf)
    for i in range(ZSTRIPE // CHUNK):
        pltpu.sync_copy(zbuf, acc.at[pl.ds(s * ZSTRIPE + i * CHUNK, CHUNK)])
    rem = ZSTRIPE % CHUNK
    if rem:
        pltpu.sync_copy(
            zbuf.at[pl.ds(0, rem)],
            acc.at[pl.ds(s * ZSTRIPE + ZSTRIPE - rem, rem)],
        )


@functools.partial(
    pl.kernel,
    out_type=jax.ShapeDtypeStruct((NC, ZROWS, DEGW), jnp.float32),
    mesh=_sc_mesh,
    scratch_types=[
        pltpu.VMEM((2, 2, CHUNK), jnp.int32),
        pltpu.VMEM((CHUNK, DEGW), jnp.float32),
        pltpu.VMEM((CHUNK, DEGW), jnp.float32),
        pltpu.VMEM_SHARED((ZROWS, DEGW), jnp.float32),
    ],
)
def _deg_kernel(idx_hbm, ones_hbm, zer_hbm, out_hbm, idxq, onesv, zbuf, acc):
    c = lax.axis_index("c")
    s = lax.axis_index("s")
    wid = c * NS + s
    _init_stripe(zer_hbm, zbuf, acc, s)
    pltpu.sync_copy(ones_hbm, onesv)
    plsc.subcore_barrier()

    def pairstep(p):
        pltpu.sync_copy(idx_hbm.at[p], idxq)
        pltpu.sync_copy(onesv, acc.at[idxq.at[0, 1]], add=True)
        pltpu.sync_copy(onesv, acc.at[idxq.at[1, 1]], add=True)

    def body(j, carry):
        pairstep(wid * PPW + j)
        return carry

    lax.fori_loop(0, PPW, body, 0)

    @pl.when(wid < 2)
    def _():
        pairstep(NW * PPW + wid)

    plsc.subcore_barrier()
    pltpu.sync_copy(
        acc.at[pl.ds(s * ZSTRIPE, ZSTRIPE)],
        out_hbm.at[c, pl.ds(s * ZSTRIPE, ZSTRIPE)],
    )


@functools.partial(
    pl.kernel,
    out_type=jax.ShapeDtypeStruct((NC, ZROWS, D), jnp.float32),
    mesh=_sc_mesh,
    scratch_types=[
        pltpu.VMEM((2, 2, CHUNK), jnp.int32),
        pltpu.VMEM((2, 2, CHUNK), jnp.int32),
        pltpu.VMEM((CHUNK, D), jnp.float32),
        pltpu.VMEM((CHUNK, D), jnp.float32),
        pltpu.VMEM_SHARED((ZROWS, D), jnp.float32),
        pltpu.SemaphoreType.DMA,
        pltpu.SemaphoreType.DMA,
        pltpu.SemaphoreType.DMA,
        pltpu.SemaphoreType.DMA,
    ],
)
def _edge_kernel(y_hbm, idx_hbm, zer_hbm, out_hbm,
                 q0, q1, rows_a, rows_b, acc, sem_a, sem_b, sem_i1, sem_i0):
    c = lax.axis_index("c")
    s = lax.axis_index("s")
    wid = c * NS + s
    _init_stripe(zer_hbm, rows_a, acc, s)
    plsc.subcore_barrier()

    p0 = wid * PPW

    def halfstep(qc, qn, pn, sem_in):
        pltpu.async_copy(y_hbm.at[qc.at[1, 0]], rows_b, sem_b)
        pltpu.async_copy(idx_hbm.at[pn], qn, sem_in)
        pltpu.make_async_copy(y_hbm.at[qc.at[0, 0]], rows_a, sem_a).wait()
        pltpu.sync_copy(rows_a, acc.at[qc.at[0, 1]], add=True)
        pltpu.make_async_copy(idx_hbm.at[pn], qn, sem_in).wait()
        pltpu.async_copy(y_hbm.at[qn.at[0, 0]], rows_a, sem_a)
        pltpu.make_async_copy(y_hbm.at[qc.at[1, 0]], rows_b, sem_b).wait()
        pltpu.sync_copy(rows_b, acc.at[qc.at[1, 1]], add=True)

    pltpu.sync_copy(idx_hbm.at[p0], q0)
    pltpu.async_copy(y_hbm.at[q0.at[0, 0]], rows_a, sem_a)

    def dbody(q, carry):
        j0 = p0 + 2 * q
        halfstep(q0, q1, j0 + 1, sem_i1)
        halfstep(q1, q0, j0 + 2, sem_i0)
        return carry

    lax.fori_loop(0, (PPW - 1) // 2, dbody, 0)

    pltpu.async_copy(y_hbm.at[q0.at[1, 0]], rows_b, sem_b)
    pltpu.make_async_copy(y_hbm.at[q0.at[0, 0]], rows_a, sem_a).wait()
    pltpu.sync_copy(rows_a, acc.at[q0.at[0, 1]], add=True)
    pltpu.make_async_copy(y_hbm.at[q0.at[1, 0]], rows_b, sem_b).wait()
    pltpu.sync_copy(rows_b, acc.at[q0.at[1, 1]], add=True)

    @pl.when(wid < 2)
    def _():
        pltpu.sync_copy(idx_hbm.at[NW * PPW + wid], q0)
        cp_a = pltpu.async_copy(y_hbm.at[q0.at[0, 0]], rows_a, sem_a)
        cp_b = pltpu.async_copy(y_hbm.at[q0.at[1, 0]], rows_b, sem_b)
        cp_a.wait()
        pltpu.sync_copy(rows_a, acc.at[q0.at[0, 1]], add=True)
        cp_b.wait()
        pltpu.sync_copy(rows_b, acc.at[q0.at[1, 1]], add=True)

    plsc.subcore_barrier()
    pltpu.sync_copy(
        acc.at[pl.ds(s * ZSTRIPE, ZSTRIPE)],
        out_hbm.at[c, pl.ds(s * ZSTRIPE, ZSTRIPE)],
    )


def _dinv_mm_body(dg_ref, x_ref, w_ref, y_ref, dinv_ref):
    dg = dg_ref[...]
    d = dg[0, :, 0:1] + dg[1, :, 0:1] + 1.0
    dinvb = jnp.broadcast_to(lax.rsqrt(d), (RBLK, D))
    xw = jnp.dot(x_ref[...], w_ref[...], preferred_element_type=jnp.float32)
    y_ref[...] = xw * dinvb
    dinv_ref[...] = dinvb


_dinv_mm = pl.pallas_call(
    _dinv_mm_body,
    grid=(GRID,),
    in_specs=[
        pl.BlockSpec((NC, RBLK, DEGW), lambda i: (0, i, 0)),
        pl.BlockSpec((RBLK, D), lambda i: (i, 0)),
        pl.BlockSpec((D, D), lambda i: (0, 0)),
    ],
    out_specs=[
        pl.BlockSpec((RBLK, D), lambda i: (i, 0)),
        pl.BlockSpec((RBLK, D), lambda i: (i, 0)),
    ],
    out_shape=[
        jax.ShapeDtypeStruct((N, D), jnp.float32),
        jax.ShapeDtypeStruct((N, D), jnp.float32),
    ],
)


def _layer2_body(z_ref, y_ref, dinv_ref, b_ref, w_ref, o_ref):
    zsum = z_ref[0] + z_ref[1]
    h = jnp.maximum(dinv_ref[...] * (zsum + y_ref[...]) + b_ref[...], 0.0)
    hw = jnp.dot(h, w_ref[...], preferred_element_type=jnp.float32)
    o_ref[...] = hw * dinv_ref[...]


_layer2 = pl.pallas_call(
    _layer2_body,
    grid=(GRID,),
    in_specs=[
        pl.BlockSpec((NC, RBLK, D), lambda i: (0, i, 0)),
        pl.BlockSpec((RBLK, D), lambda i: (i, 0)),
        pl.BlockSpec((RBLK, D), lambda i: (i, 0)),
        pl.BlockSpec((D,), lambda i: (0,)),
        pl.BlockSpec((D, D), lambda i: (0, 0)),
    ],
    out_specs=pl.BlockSpec((RBLK, D), lambda i: (i, 0)),
    out_shape=jax.ShapeDtypeStruct((N, D), jnp.float32),
)


def _final_body(z_ref, y_ref, dinv_ref, b_ref, o_ref):
    o = dinv_ref[...] * (z_ref[0] + z_ref[1] + y_ref[...]) + b_ref[...]
    m = jnp.max(o, axis=1, keepdims=True)
    t = o - m
    o_ref[...] = t - jnp.log(jnp.sum(jnp.exp(t), axis=1, keepdims=True))


_final = pl.pallas_call(
    _final_body,
    grid=(GRID,),
    in_specs=[
        pl.BlockSpec((NC, RBLK, D), lambda i: (0, i, 0)),
        pl.BlockSpec((RBLK, D), lambda i: (i, 0)),
        pl.BlockSpec((RBLK, D), lambda i: (i, 0)),
        pl.BlockSpec((D,), lambda i: (0,)),
    ],
    out_specs=pl.BlockSpec((RBLK, D), lambda i: (i, 0)),
    out_shape=jax.ShapeDtypeStruct((N, D), jnp.float32),
)


def kernel(x, edge_index, W1, b1, W2, b2):
    src = edge_index[0].astype(jnp.int32).reshape(PAIRS, 2, 1, CHUNK)
    dst = edge_index[1].astype(jnp.int32).reshape(PAIRS, 2, 1, CHUNK)
    idxp = jnp.concatenate([src, dst], axis=2)
    ones_rows = jnp.ones((CHUNK, DEGW), jnp.float32)
    zer_d = jnp.zeros((CHUNK, D), jnp.float32)

    degp = _deg_kernel(idxp, ones_rows, zer_d)
    y1, dinvb = _dinv_mm(degp, x, W1)
    z1 = _edge_kernel(y1, idxp, zer_d)
    y2 = _layer2(z1, y1, dinvb, b1, W2)
    z2 = _edge_kernel(y2, idxp, zer_d)
    return _final(z2, y2, dinvb, b2)


# speedup vs baseline: 27.2632x; 1.0103x over previous
import functools

import jax
import jax.numpy as jnp
from jax import lax
from jax.experimental import pallas as pl
from jax.experimental.pallas import tpu as pltpu
from jax.experimental.pallas import tpu_sc as plsc

N = 10000
E = 320000
D = 128
NC = 2
NS = 16
NW = NC * NS
CHUNK = 128
ROWS = E // CHUNK
PAIRS = ROWS // 2
PPW = PAIRS // NW
ZROWS = 10112
ZSTRIPE = ZROWS // NS
DEGW = 128
RBLK = 1000
GRID = N // RBLK

_sc_mesh = plsc.VectorSubcoreMesh(
    core_axis_name="c", subcore_axis_name="s", num_cores=NC, num_subcores=NS
)


def _init_stripe(zer_hbm, zbuf, acc, s):
    pltpu.sync_copy(zer_hbm, zbuf)
    for i in range(ZSTRIPE // CHUNK):
        pltpu.sync_copy(zbuf, acc.at[pl.ds(s * ZSTRIPE + i * CHUNK, CHUNK)])
    rem = ZSTRIPE % CHUNK
    if rem:
        pltpu.sync_copy(
            zbuf.at[pl.ds(0, rem)],
            acc.at[pl.ds(s * ZSTRIPE + ZSTRIPE - rem, rem)],
        )


@functools.partial(
    pl.kernel,
    out_type=jax.ShapeDtypeStruct((NC, ZROWS, DEGW), jnp.float32),
    mesh=_sc_mesh,
    scratch_types=[
        pltpu.VMEM((2, CHUNK), jnp.int32),
        pltpu.VMEM((CHUNK, DEGW), jnp.float32),
        pltpu.VMEM((CHUNK, DEGW), jnp.float32),
        pltpu.VMEM_SHARED((ZROWS, DEGW), jnp.float32),
    ],
)
def _deg_kernel(idx_hbm, ones_hbm, zer_hbm, out_hbm, idxq, onesv, zbuf, acc):
    c = lax.axis_index("c")
    s = lax.axis_index("s")
    wid = c * NS + s
    _init_stripe(zer_hbm, zbuf, acc, s)
    pltpu.sync_copy(ones_hbm, onesv)
    plsc.subcore_barrier()

    def pairstep(p):
        pltpu.sync_copy(idx_hbm.at[1, p], idxq)
        pltpu.sync_copy(onesv, acc.at[idxq.at[0]], add=True)
        pltpu.sync_copy(onesv, acc.at[idxq.at[1]], add=True)

    def body(j, carry):
        pairstep(wid * PPW + j)
        return carry

    lax.fori_loop(0, PPW, body, 0)

    @pl.when(wid < 2)
    def _():
        pairstep(NW * PPW + wid)

    plsc.subcore_barrier()
    pltpu.sync_copy(
        acc.at[pl.ds(s * ZSTRIPE, ZSTRIPE)],
        out_hbm.at[c, pl.ds(s * ZSTRIPE, ZSTRIPE)],
    )


@functools.partial(
    pl.kernel,
    out_type=jax.ShapeDtypeStruct((NC, ZROWS, D), jnp.float32),
    mesh=_sc_mesh,
    scratch_types=[
        pltpu.VMEM((2, 2, CHUNK), jnp.int32),
        pltpu.VMEM((2, 2, CHUNK), jnp.int32),
        pltpu.VMEM((CHUNK, D), jnp.float32),
        pltpu.VMEM((CHUNK, D), jnp.float32),
        pltpu.VMEM_SHARED((ZROWS, D), jnp.float32),
        pltpu.SemaphoreType.DMA,
        pltpu.SemaphoreType.DMA,
        pltpu.SemaphoreType.DMA,
        pltpu.SemaphoreType.DMA,
    ],
)
def _edge_kernel(y_hbm, idx_hbm, zer_hbm, out_hbm,
                 q0, q1, rows_a, rows_b, acc, sem_a, sem_b, sem_i1, sem_i0):
    c = lax.axis_index("c")
    s = lax.axis_index("s")
    wid = c * NS + s
    _init_stripe(zer_hbm, rows_a, acc, s)
    plsc.subcore_barrier()

    p0 = wid * PPW

    def halfstep(qc, qn, pn, sem_in):
        pltpu.async_copy(y_hbm.at[qc.at[0, 1]], rows_b, sem_b)
        pltpu.async_copy(idx_hbm.at[0, pn], qn.at[0], sem_in)
        pltpu.async_copy(idx_hbm.at[1, pn], qn.at[1], sem_in)
        pltpu.make_async_copy(y_hbm.at[qc.at[0, 0]], rows_a, sem_a).wait()
        pltpu.sync_copy(rows_a, acc.at[qc.at[1, 0]], add=True)
        pltpu.make_async_copy(idx_hbm.at[0, pn], qn.at[0], sem_in).wait()
        pltpu.make_async_copy(idx_hbm.at[1, pn], qn.at[1], sem_in).wait()
        pltpu.async_copy(y_hbm.at[qn.at[0, 0]], rows_a, sem_a)
        pltpu.make_async_copy(y_hbm.at[qc.at[0, 1]], rows_b, sem_b).wait()
        pltpu.sync_copy(rows_b, acc.at[qc.at[1, 1]], add=True)

    pltpu.sync_copy(idx_hbm.at[0, p0], q0.at[0])
    pltpu.sync_copy(idx_hbm.at[1, p0], q0.at[1])
    pltpu.async_copy(y_hbm.at[q0.at[0, 0]], rows_a, sem_a)

    def dbody(q, carry):
        j0 = p0 + 2 * q
        halfstep(q0, q1, j0 + 1, sem_i1)
        halfstep(q1, q0, j0 + 2, sem_i0)
        return carry

    lax.fori_loop(0, (PPW - 1) // 2, dbody, 0)

    pltpu.async_copy(y_hbm.at[q0.at[0, 1]], rows_b, sem_b)
    pltpu.make_async_copy(y_hbm.at[q0.at[0, 0]], rows_a, sem_a).wait()
    pltpu.sync_copy(rows_a, acc.at[q0.at[1, 0]], add=True)
    pltpu.make_async_copy(y_hbm.at[q0.at[0, 1]], rows_b, sem_b).wait()
    pltpu.sync_copy(rows_b, acc.at[q0.at[1, 1]], add=True)

    @pl.when(wid < 2)
    def _():
        pltpu.sync_copy(idx_hbm.at[0, NW * PPW + wid], q0.at[0])
        pltpu.sync_copy(idx_hbm.at[1, NW * PPW + wid], q0.at[1])
        cp_a = pltpu.async_copy(y_hbm.at[q0.at[0, 0]], rows_a, sem_a)
        cp_b = pltpu.async_copy(y_hbm.at[q0.at[0, 1]], rows_b, sem_b)
        cp_a.wait()
        pltpu.sync_copy(rows_a, acc.at[q0.at[1, 0]], add=True)
        cp_b.wait()
        pltpu.sync_copy(rows_b, acc.at[q0.at[1, 1]], add=True)

    plsc.subcore_barrier()
    pltpu.sync_copy(
        acc.at[pl.ds(s * ZSTRIPE, ZSTRIPE)],
        out_hbm.at[c, pl.ds(s * ZSTRIPE, ZSTRIPE)],
    )


def _dinv_mm_body(dg_ref, x_ref, w_ref, y_ref, dinv_ref):
    dg = dg_ref[...]
    d = dg[0, :, 0:1] + dg[1, :, 0:1] + 1.0
    dinvb = jnp.broadcast_to(lax.rsqrt(d), (RBLK, D))
    xw = jnp.dot(x_ref[...], w_ref[...], preferred_element_type=jnp.float32)
    y_ref[...] = xw * dinvb
    dinv_ref[...] = dinvb


_dinv_mm = pl.pallas_call(
    _dinv_mm_body,
    grid=(GRID,),
    in_specs=[
        pl.BlockSpec((NC, RBLK, DEGW), lambda i: (0, i, 0)),
        pl.BlockSpec((RBLK, D), lambda i: (i, 0)),
        pl.BlockSpec((D, D), lambda i: (0, 0)),
    ],
    out_specs=[
        pl.BlockSpec((RBLK, D), lambda i: (i, 0)),
        pl.BlockSpec((RBLK, D), lambda i: (i, 0)),
    ],
    out_shape=[
        jax.ShapeDtypeStruct((N, D), jnp.float32),
        jax.ShapeDtypeStruct((N, D), jnp.float32),
    ],
)


def _layer2_body(z_ref, y_ref, dinv_ref, b_ref, w_ref, o_ref):
    zsum = z_ref[0] + z_ref[1]
    h = jnp.maximum(dinv_ref[...] * (zsum + y_ref[...]) + b_ref[...], 0.0)
    hw = jnp.dot(h, w_ref[...], preferred_element_type=jnp.float32)
    o_ref[...] = hw * dinv_ref[...]


_layer2 = pl.pallas_call(
    _layer2_body,
    grid=(GRID,),
    in_specs=[
        pl.BlockSpec((NC, RBLK, D), lambda i: (0, i, 0)),
        pl.BlockSpec((RBLK, D), lambda i: (i, 0)),
        pl.BlockSpec((RBLK, D), lambda i: (i, 0)),
        pl.BlockSpec((D,), lambda i: (0,)),
        pl.BlockSpec((D, D), lambda i: (0, 0)),
    ],
    out_specs=pl.BlockSpec((RBLK, D), lambda i: (i, 0)),
    out_shape=jax.ShapeDtypeStruct((N, D), jnp.float32),
)


def _final_body(z_ref, y_ref, dinv_ref, b_ref, o_ref):
    o = dinv_ref[...] * (z_ref[0] + z_ref[1] + y_ref[...]) + b_ref[...]
    m = jnp.max(o, axis=1, keepdims=True)
    t = o - m
    o_ref[...] = t - jnp.log(jnp.sum(jnp.exp(t), axis=1, keepdims=True))


_final = pl.pallas_call(
    _final_body,
    grid=(GRID,),
    in_specs=[
        pl.BlockSpec((NC, RBLK, D), lambda i: (0, i, 0)),
        pl.BlockSpec((RBLK, D), lambda i: (i, 0)),
        pl.BlockSpec((RBLK, D), lambda i: (i, 0)),
        pl.BlockSpec((D,), lambda i: (0,)),
    ],
    out_specs=pl.BlockSpec((RBLK, D), lambda i: (i, 0)),
    out_shape=jax.ShapeDtypeStruct((N, D), jnp.float32),
)


def kernel(x, edge_index, W1, b1, W2, b2):
    idxp = edge_index.astype(jnp.int32).reshape(2, PAIRS, 2, CHUNK)
    ones_rows = jnp.ones((CHUNK, DEGW), jnp.float32)
    zer_d = jnp.zeros((CHUNK, D), jnp.float32)

    degp = _deg_kernel(idxp, ones_rows, zer_d)
    y1, dinvb = _dinv_mm(degp, x, W1)
    z1 = _edge_kernel(y1, idxp, zer_d)
    y2 = _layer2(z1, y1, dinvb, b1, W2)
    z2 = _edge_kernel(y2, idxp, zer_d)
    return _final(z2, y2, dinvb, b2)


# speedup vs baseline: 28.2638x; 1.0367x over previous
import functools

import jax
import jax.numpy as jnp
from jax import lax
from jax.experimental import pallas as pl
from jax.experimental.pallas import tpu as pltpu
from jax.experimental.pallas import tpu_sc as plsc

N = 10000
E = 320000
D = 128
NC = 2
NS = 16
NW = NC * NS
CHUNK = 128
ROWS = E // CHUNK
PAIRS = ROWS // 2
PPW = PAIRS // NW
ZROWS = 10112
ZSTRIPE = ZROWS // NS
DEGW = 128
RBLK = 1000
GRID = N // RBLK

_sc_mesh = plsc.VectorSubcoreMesh(
    core_axis_name="c", subcore_axis_name="s", num_cores=NC, num_subcores=NS
)


def _init_stripe(zer_hbm, zbuf, acc, s):
    pltpu.sync_copy(zer_hbm, zbuf)
    for i in range(ZSTRIPE // CHUNK):
        pltpu.sync_copy(zbuf, acc.at[pl.ds(s * ZSTRIPE + i * CHUNK, CHUNK)])
    rem = ZSTRIPE % CHUNK
    if rem:
        pltpu.sync_copy(
            zbuf.at[pl.ds(0, rem)],
            acc.at[pl.ds(s * ZSTRIPE + ZSTRIPE - rem, rem)],
        )


@functools.partial(
    pl.kernel,
    out_type=jax.ShapeDtypeStruct((NC, ZROWS, DEGW), jnp.float32),
    mesh=_sc_mesh,
    scratch_types=[
        pltpu.VMEM((2, CHUNK), jnp.int32),
        pltpu.VMEM((2, CHUNK), jnp.int32),
        pltpu.VMEM((CHUNK, DEGW), jnp.float32),
        pltpu.VMEM((CHUNK, DEGW), jnp.float32),
        pltpu.VMEM_SHARED((ZROWS, DEGW), jnp.float32),
        pltpu.SemaphoreType.DMA,
    ],
)
def _deg_kernel(idx_hbm, ones_hbm, zer_hbm, out_hbm,
                q0, q1, onesv, zbuf, acc, sem_i):
    c = lax.axis_index("c")
    s = lax.axis_index("s")
    wid = c * NS + s
    _init_stripe(zer_hbm, zbuf, acc, s)
    pltpu.sync_copy(ones_hbm, onesv)
    plsc.subcore_barrier()

    p0 = wid * PPW

    def load(p, q):
        pltpu.async_copy(idx_hbm.at[1, pl.ds(p * 2 * CHUNK, CHUNK)],
                         q.at[0], sem_i)
        pltpu.async_copy(idx_hbm.at[1, pl.ds(p * 2 * CHUNK + CHUNK, CHUNK)],
                         q.at[1], sem_i)

    def drain(p, q):
        pltpu.make_async_copy(idx_hbm.at[1, pl.ds(p * 2 * CHUNK, CHUNK)],
                              q.at[0], sem_i).wait()
        pltpu.make_async_copy(idx_hbm.at[1, pl.ds(p * 2 * CHUNK, CHUNK)],
                              q.at[1], sem_i).wait()

    def scat(q):
        pltpu.sync_copy(onesv, acc.at[q.at[0]], add=True)
        pltpu.sync_copy(onesv, acc.at[q.at[1]], add=True)

    load(p0, q0)
    drain(p0, q0)

    def dbody(k, carry):
        j0 = p0 + 2 * k
        load(j0 + 1, q1)
        scat(q0)
        drain(j0 + 1, q1)
        load(j0 + 2, q0)
        scat(q1)
        drain(j0 + 2, q0)
        return carry

    lax.fori_loop(0, (PPW - 1) // 2, dbody, 0)
    scat(q0)

    @pl.when(wid < 2)
    def _():
        load(NW * PPW + wid, q1)
        drain(NW * PPW + wid, q1)
        scat(q1)

    plsc.subcore_barrier()
    pltpu.sync_copy(
        acc.at[pl.ds(s * ZSTRIPE, ZSTRIPE)],
        out_hbm.at[c, pl.ds(s * ZSTRIPE, ZSTRIPE)],
    )


@functools.partial(
    pl.kernel,
    out_type=jax.ShapeDtypeStruct((NC, ZROWS, D), jnp.float32),
    mesh=_sc_mesh,
    scratch_types=[
        pltpu.VMEM((2, 2, CHUNK), jnp.int32),
        pltpu.VMEM((2, 2, CHUNK), jnp.int32),
        pltpu.VMEM((CHUNK, D), jnp.float32),
        pltpu.VMEM((CHUNK, D), jnp.float32),
        pltpu.VMEM_SHARED((ZROWS, D), jnp.float32),
        pltpu.SemaphoreType.DMA,
        pltpu.SemaphoreType.DMA,
        pltpu.SemaphoreType.DMA,
        pltpu.SemaphoreType.DMA,
    ],
)
def _edge_kernel(y_hbm, idx_hbm, zer_hbm, out_hbm,
                 q0, q1, rows_a, rows_b, acc, sem_a, sem_b, sem_i1, sem_i0):
    c = lax.axis_index("c")
    s = lax.axis_index("s")
    wid = c * NS + s
    _init_stripe(zer_hbm, rows_a, acc, s)
    plsc.subcore_barrier()

    p0 = wid * PPW

    def halfstep(qc, qn, pn, sem_in):
        pltpu.async_copy(y_hbm.at[qc.at[0, 1]], rows_b, sem_b)
        pltpu.async_copy(idx_hbm.at[0, pl.ds(pn * 2 * CHUNK, CHUNK)],
                         qn.at[0, 0], sem_in)
        pltpu.async_copy(idx_hbm.at[0, pl.ds(pn * 2 * CHUNK + CHUNK, CHUNK)],
                         qn.at[0, 1], sem_in)
        pltpu.async_copy(idx_hbm.at[1, pl.ds(pn * 2 * CHUNK, CHUNK)],
                         qn.at[1, 0], sem_in)
        pltpu.async_copy(idx_hbm.at[1, pl.ds(pn * 2 * CHUNK + CHUNK, CHUNK)],
                         qn.at[1, 1], sem_in)
        pltpu.make_async_copy(y_hbm.at[qc.at[0, 0]], rows_a, sem_a).wait()
        pltpu.sync_copy(rows_a, acc.at[qc.at[1, 0]], add=True)
        for _k in range(4):
            pltpu.make_async_copy(idx_hbm.at[0, pl.ds(pn * 2 * CHUNK, CHUNK)],
                                  qn.at[0, 0], sem_in).wait()
        pltpu.async_copy(y_hbm.at[qn.at[0, 0]], rows_a, sem_a)
        pltpu.make_async_copy(y_hbm.at[qc.at[0, 1]], rows_b, sem_b).wait()
        pltpu.sync_copy(rows_b, acc.at[qc.at[1, 1]], add=True)

    pltpu.sync_copy(idx_hbm.at[0, pl.ds(p0 * 2 * CHUNK, CHUNK)], q0.at[0, 0])
    pltpu.sync_copy(idx_hbm.at[0, pl.ds(p0 * 2 * CHUNK + CHUNK, CHUNK)], q0.at[0, 1])
    pltpu.sync_copy(idx_hbm.at[1, pl.ds(p0 * 2 * CHUNK, CHUNK)], q0.at[1, 0])
    pltpu.sync_copy(idx_hbm.at[1, pl.ds(p0 * 2 * CHUNK + CHUNK, CHUNK)], q0.at[1, 1])
    pltpu.async_copy(y_hbm.at[q0.at[0, 0]], rows_a, sem_a)

    def dbody(q, carry):
        j0 = p0 + 2 * q
        halfstep(q0, q1, j0 + 1, sem_i1)
        halfstep(q1, q0, j0 + 2, sem_i0)
        return carry

    lax.fori_loop(0, (PPW - 1) // 2, dbody, 0)

    pltpu.async_copy(y_hbm.at[q0.at[0, 1]], rows_b, sem_b)
    pltpu.make_async_copy(y_hbm.at[q0.at[0, 0]], rows_a, sem_a).wait()
    pltpu.sync_copy(rows_a, acc.at[q0.at[1, 0]], add=True)
    pltpu.make_async_copy(y_hbm.at[q0.at[0, 1]], rows_b, sem_b).wait()
    pltpu.sync_copy(rows_b, acc.at[q0.at[1, 1]], add=True)

    @pl.when(wid < 2)
    def _():
        pe = NW * PPW + wid
        pltpu.sync_copy(idx_hbm.at[0, pl.ds(pe * 2 * CHUNK, CHUNK)], q0.at[0, 0])
        pltpu.sync_copy(idx_hbm.at[0, pl.ds(pe * 2 * CHUNK + CHUNK, CHUNK)], q0.at[0, 1])
        pltpu.sync_copy(idx_hbm.at[1, pl.ds(pe * 2 * CHUNK, CHUNK)], q0.at[1, 0])
        pltpu.sync_copy(idx_hbm.at[1, pl.ds(pe * 2 * CHUNK + CHUNK, CHUNK)], q0.at[1, 1])
        cp_a = pltpu.async_copy(y_hbm.at[q0.at[0, 0]], rows_a, sem_a)
        cp_b = pltpu.async_copy(y_hbm.at[q0.at[0, 1]], rows_b, sem_b)
        cp_a.wait()
        pltpu.sync_copy(rows_a, acc.at[q0.at[1, 0]], add=True)
        cp_b.wait()
        pltpu.sync_copy(rows_b, acc.at[q0.at[1, 1]], add=True)

    plsc.subcore_barrier()
    pltpu.sync_copy(
        acc.at[pl.ds(s * ZSTRIPE, ZSTRIPE)],
        out_hbm.at[c, pl.ds(s * ZSTRIPE, ZSTRIPE)],
    )


def _dinv_mm_body(dg_ref, x_ref, w_ref, y_ref, dinv_ref):
    dg = dg_ref[...]
    d = dg[0, :, 0:1] + dg[1, :, 0:1] + 1.0
    dinvb = jnp.broadcast_to(lax.rsqrt(d), (RBLK, D))
    xw = jnp.dot(x_ref[...], w_ref[...], preferred_element_type=jnp.float32)
    y_ref[...] = xw * dinvb
    dinv_ref[...] = dinvb


_dinv_mm = pl.pallas_call(
    _dinv_mm_body,
    grid=(GRID,),
    in_specs=[
        pl.BlockSpec((NC, RBLK, DEGW), lambda i: (0, i, 0)),
        pl.BlockSpec((RBLK, D), lambda i: (i, 0)),
        pl.BlockSpec((D, D), lambda i: (0, 0)),
    ],
    out_specs=[
        pl.BlockSpec((RBLK, D), lambda i: (i, 0)),
        pl.BlockSpec((RBLK, D), lambda i: (i, 0)),
    ],
    out_shape=[
        jax.ShapeDtypeStruct((N, D), jnp.float32),
        jax.ShapeDtypeStruct((N, D), jnp.float32),
    ],
)


def _layer2_body(z_ref, y_ref, dinv_ref, b_ref, w_ref, o_ref):
    zsum = z_ref[0] + z_ref[1]
    h = jnp.maximum(dinv_ref[...] * (zsum + y_ref[...]) + b_ref[...], 0.0)
    hw = jnp.dot(h, w_ref[...], preferred_element_type=jnp.float32)
    o_ref[...] = hw * dinv_ref[...]


_layer2 = pl.pallas_call(
    _layer2_body,
    grid=(GRID,),
    in_specs=[
        pl.BlockSpec((NC, RBLK, D), lambda i: (0, i, 0)),
        pl.BlockSpec((RBLK, D), lambda i: (i, 0)),
        pl.BlockSpec((RBLK, D), lambda i: (i, 0)),
        pl.BlockSpec((D,), lambda i: (0,)),
        pl.BlockSpec((D, D), lambda i: (0, 0)),
    ],
    out_specs=pl.BlockSpec((RBLK, D), lambda i: (i, 0)),
    out_shape=jax.ShapeDtypeStruct((N, D), jnp.float32),
)


def _final_body(z_ref, y_ref, dinv_ref, b_ref, o_ref):
    o = dinv_ref[...] * (z_ref[0] + z_ref[1] + y_ref[...]) + b_ref[...]
    m = jnp.max(o, axis=1, keepdims=True)
    t = o - m
    o_ref[...] = t - jnp.log(jnp.sum(jnp.exp(t), axis=1, keepdims=True))


_final = pl.pallas_call(
    _final_body,
    grid=(GRID,),
    in_specs=[
        pl.BlockSpec((NC, RBLK, D), lambda i: (0, i, 0)),
        pl.BlockSpec((RBLK, D), lambda i: (i, 0)),
        pl.BlockSpec((RBLK, D), lambda i: (i, 0)),
        pl.BlockSpec((D,), lambda i: (0,)),
    ],
    out_specs=pl.BlockSpec((RBLK, D), lambda i: (i, 0)),
    out_shape=jax.ShapeDtypeStruct((N, D), jnp.float32),
)


def kernel(x, edge_index, W1, b1, W2, b2):
    idxp = edge_index.astype(jnp.int32)
    ones_rows = jnp.ones((CHUNK, DEGW), jnp.float32)
    zer_d = jnp.zeros((CHUNK, D), jnp.float32)

    degp = _deg_kernel(idxp, ones_rows, zer_d)
    y1, dinvb = _dinv_mm(degp, x, W1)
    z1 = _edge_kernel(y1, idxp, zer_d)
    y2 = _layer2(z1, y1, dinvb, b1, W2)
    z2 = _edge_kernel(y2, idxp, zer_d)
    return _final(z2, y2, dinvb, b2)


# speedup vs baseline: 28.7673x; 1.0178x over previous
import functools

import jax
import jax.numpy as jnp
from jax import lax
from jax.experimental import pallas as pl
from jax.experimental.pallas import tpu as pltpu
from jax.experimental.pallas import tpu_sc as plsc

N = 10000
E = 320000
D = 128
NC = 2
NS = 16
NW = NC * NS
CHUNK = 128
ROWS = E // CHUNK
PAIRS = ROWS // 2
PPW = PAIRS // NW
ZROWS = 10112
ZSTRIPE = ZROWS // NS
DEGW = 128
RBLK = 2000
GRID = N // RBLK

_sc_mesh = plsc.VectorSubcoreMesh(
    core_axis_name="c", subcore_axis_name="s", num_cores=NC, num_subcores=NS
)


def _init_stripe(zer_hbm, zbuf, acc, s):
    pltpu.sync_copy(zer_hbm, zbuf)
    for i in range(ZSTRIPE // CHUNK):
        pltpu.sync_copy(zbuf, acc.at[pl.ds(s * ZSTRIPE + i * CHUNK, CHUNK)])
    rem = ZSTRIPE % CHUNK
    if rem:
        pltpu.sync_copy(
            zbuf.at[pl.ds(0, rem)],
            acc.at[pl.ds(s * ZSTRIPE + ZSTRIPE - rem, rem)],
        )


@functools.partial(
    pl.kernel,
    out_type=jax.ShapeDtypeStruct((NC, ZROWS, DEGW), jnp.float32),
    mesh=_sc_mesh,
    scratch_types=[
        pltpu.VMEM((2, CHUNK), jnp.int32),
        pltpu.VMEM((2, CHUNK), jnp.int32),
        pltpu.VMEM((CHUNK, DEGW), jnp.float32),
        pltpu.VMEM((CHUNK, DEGW), jnp.float32),
        pltpu.VMEM_SHARED((ZROWS, DEGW), jnp.float32),
        pltpu.SemaphoreType.DMA,
    ],
)
def _deg_kernel(idx_hbm, ones_hbm, zer_hbm, out_hbm,
                q0, q1, onesv, zbuf, acc, sem_i):
    c = lax.axis_index("c")
    s = lax.axis_index("s")
    wid = c * NS + s
    _init_stripe(zer_hbm, zbuf, acc, s)
    pltpu.sync_copy(ones_hbm, onesv)
    plsc.subcore_barrier()

    p0 = wid * PPW

    def load(p, q):
        pltpu.async_copy(idx_hbm.at[1, pl.ds(p * 2 * CHUNK, CHUNK)],
                         q.at[0], sem_i)
        pltpu.async_copy(idx_hbm.at[1, pl.ds(p * 2 * CHUNK + CHUNK, CHUNK)],
                         q.at[1], sem_i)

    def drain(p, q):
        pltpu.make_async_copy(idx_hbm.at[1, pl.ds(p * 2 * CHUNK, CHUNK)],
                              q.at[0], sem_i).wait()
        pltpu.make_async_copy(idx_hbm.at[1, pl.ds(p * 2 * CHUNK, CHUNK)],
                              q.at[1], sem_i).wait()

    def scat(q):
        pltpu.sync_copy(onesv, acc.at[q.at[0]], add=True)
        pltpu.sync_copy(onesv, acc.at[q.at[1]], add=True)

    load(p0, q0)
    drain(p0, q0)

    def dbody(k, carry):
        j0 = p0 + 2 * k
        load(j0 + 1, q1)
        scat(q0)
        drain(j0 + 1, q1)
        load(j0 + 2, q0)
        scat(q1)
        drain(j0 + 2, q0)
        return carry

    lax.fori_loop(0, (PPW - 1) // 2, dbody, 0)
    scat(q0)

    @pl.when(wid < 2)
    def _():
        load(NW * PPW + wid, q1)
        drain(NW * PPW + wid, q1)
        scat(q1)

    plsc.subcore_barrier()
    pltpu.sync_copy(
        acc.at[pl.ds(s * ZSTRIPE, ZSTRIPE)],
        out_hbm.at[c, pl.ds(s * ZSTRIPE, ZSTRIPE)],
    )


@functools.partial(
    pl.kernel,
    out_type=jax.ShapeDtypeStruct((NC, ZROWS, D), jnp.float32),
    mesh=_sc_mesh,
    scratch_types=[
        pltpu.VMEM((2, 2, CHUNK), jnp.int32),
        pltpu.VMEM((2, 2, CHUNK), jnp.int32),
        pltpu.VMEM((CHUNK, D), jnp.float32),
        pltpu.VMEM((CHUNK, D), jnp.float32),
        pltpu.VMEM_SHARED((ZROWS, D), jnp.float32),
        pltpu.SemaphoreType.DMA,
        pltpu.SemaphoreType.DMA,
        pltpu.SemaphoreType.DMA,
        pltpu.SemaphoreType.DMA,
    ],
)
def _edge_kernel(y_hbm, idx_hbm, zer_hbm, out_hbm,
                 q0, q1, rows_a, rows_b, acc, sem_a, sem_b, sem_i1, sem_i0):
    c = lax.axis_index("c")
    s = lax.axis_index("s")
    wid = c * NS + s
    _init_stripe(zer_hbm, rows_a, acc, s)
    plsc.subcore_barrier()

    p0 = wid * PPW

    def halfstep(qc, qn, pn, sem_in):
        pltpu.async_copy(y_hbm.at[qc.at[0, 1]], rows_b, sem_b)
        pltpu.async_copy(idx_hbm.at[0, pl.ds(pn * 2 * CHUNK, CHUNK)],
                         qn.at[0, 0], sem_in)
        pltpu.async_copy(idx_hbm.at[0, pl.ds(pn * 2 * CHUNK + CHUNK, CHUNK)],
                         qn.at[0, 1], sem_in)
        pltpu.async_copy(idx_hbm.at[1, pl.ds(pn * 2 * CHUNK, CHUNK)],
                         qn.at[1, 0], sem_in)
        pltpu.async_copy(idx_hbm.at[1, pl.ds(pn * 2 * CHUNK + CHUNK, CHUNK)],
                         qn.at[1, 1], sem_in)
        pltpu.make_async_copy(y_hbm.at[qc.at[0, 0]], rows_a, sem_a).wait()
        pltpu.sync_copy(rows_a, acc.at[qc.at[1, 0]], add=True)
        for _k in range(4):
            pltpu.make_async_copy(idx_hbm.at[0, pl.ds(pn * 2 * CHUNK, CHUNK)],
                                  qn.at[0, 0], sem_in).wait()
        pltpu.async_copy(y_hbm.at[qn.at[0, 0]], rows_a, sem_a)
        pltpu.make_async_copy(y_hbm.at[qc.at[0, 1]], rows_b, sem_b).wait()
        pltpu.sync_copy(rows_b, acc.at[qc.at[1, 1]], add=True)

    pltpu.sync_copy(idx_hbm.at[0, pl.ds(p0 * 2 * CHUNK, CHUNK)], q0.at[0, 0])
    pltpu.sync_copy(idx_hbm.at[0, pl.ds(p0 * 2 * CHUNK + CHUNK, CHUNK)], q0.at[0, 1])
    pltpu.sync_copy(idx_hbm.at[1, pl.ds(p0 * 2 * CHUNK, CHUNK)], q0.at[1, 0])
    pltpu.sync_copy(idx_hbm.at[1, pl.ds(p0 * 2 * CHUNK + CHUNK, CHUNK)], q0.at[1, 1])
    pltpu.async_copy(y_hbm.at[q0.at[0, 0]], rows_a, sem_a)

    def dbody(q, carry):
        j0 = p0 + 2 * q
        halfstep(q0, q1, j0 + 1, sem_i1)
        halfstep(q1, q0, j0 + 2, sem_i0)
        return carry

    lax.fori_loop(0, (PPW - 1) // 2, dbody, 0)

    pltpu.async_copy(y_hbm.at[q0.at[0, 1]], rows_b, sem_b)
    pltpu.make_async_copy(y_hbm.at[q0.at[0, 0]], rows_a, sem_a).wait()
    pltpu.sync_copy(rows_a, acc.at[q0.at[1, 0]], add=True)
    pltpu.make_async_copy(y_hbm.at[q0.at[0, 1]], rows_b, sem_b).wait()
    pltpu.sync_copy(rows_b, acc.at[q0.at[1, 1]], add=True)

    @pl.when(wid < 2)
    def _():
        pe = NW * PPW + wid
        pltpu.sync_copy(idx_hbm.at[0, pl.ds(pe * 2 * CHUNK, CHUNK)], q0.at[0, 0])
        pltpu.sync_copy(idx_hbm.at[0, pl.ds(pe * 2 * CHUNK + CHUNK, CHUNK)], q0.at[0, 1])
        pltpu.sync_copy(idx_hbm.at[1, pl.ds(pe * 2 * CHUNK, CHUNK)], q0.at[1, 0])
        pltpu.sync_copy(idx_hbm.at[1, pl.ds(pe * 2 * CHUNK + CHUNK, CHUNK)], q0.at[1, 1])
        cp_a = pltpu.async_copy(y_hbm.at[q0.at[0, 0]], rows_a, sem_a)
        cp_b = pltpu.async_copy(y_hbm.at[q0.at[0, 1]], rows_b, sem_b)
        cp_a.wait()
        pltpu.sync_copy(rows_a, acc.at[q0.at[1, 0]], add=True)
        cp_b.wait()
        pltpu.sync_copy(rows_b, acc.at[q0.at[1, 1]], add=True)

    plsc.subcore_barrier()
    pltpu.sync_copy(
        acc.at[pl.ds(s * ZSTRIPE, ZSTRIPE)],
        out_hbm.at[c, pl.ds(s * ZSTRIPE, ZSTRIPE)],
    )


def _dinv_mm_body(dg_ref, x_ref, w_ref, y_ref, dinv_ref):
    dg = dg_ref[...]
    d = dg[0, :, 0:1] + dg[1, :, 0:1] + 1.0
    dinvb = jnp.broadcast_to(lax.rsqrt(d), (RBLK, D))
    xw = jnp.dot(x_ref[...], w_ref[...], preferred_element_type=jnp.float32)
    y_ref[...] = xw * dinvb
    dinv_ref[...] = dinvb


_dinv_mm = pl.pallas_call(
    _dinv_mm_body,
    grid=(GRID,),
    in_specs=[
        pl.BlockSpec((NC, RBLK, DEGW), lambda i: (0, i, 0)),
        pl.BlockSpec((RBLK, D), lambda i: (i, 0)),
        pl.BlockSpec((D, D), lambda i: (0, 0)),
    ],
    out_specs=[
        pl.BlockSpec((RBLK, D), lambda i: (i, 0)),
        pl.BlockSpec((RBLK, D), lambda i: (i, 0)),
    ],
    out_shape=[
        jax.ShapeDtypeStruct((N, D), jnp.float32),
        jax.ShapeDtypeStruct((N, D), jnp.float32),
    ],
)


def _layer2_body(z_ref, y_ref, dinv_ref, b_ref, w_ref, o_ref):
    zsum = z_ref[0] + z_ref[1]
    h = jnp.maximum(dinv_ref[...] * (zsum + y_ref[...]) + b_ref[...], 0.0)
    hw = jnp.dot(h, w_ref[...], preferred_element_type=jnp.float32)
    o_ref[...] = hw * dinv_ref[...]


_layer2 = pl.pallas_call(
    _layer2_body,
    grid=(GRID,),
    in_specs=[
        pl.BlockSpec((NC, RBLK, D), lambda i: (0, i, 0)),
        pl.BlockSpec((RBLK, D), lambda i: (i, 0)),
        pl.BlockSpec((RBLK, D), lambda i: (i, 0)),
        pl.BlockSpec((D,), lambda i: (0,)),
        pl.BlockSpec((D, D), lambda i: (0, 0)),
    ],
    out_specs=pl.BlockSpec((RBLK, D), lambda i: (i, 0)),
    out_shape=jax.ShapeDtypeStruct((N, D), jnp.float32),
)


def _final_body(z_ref, y_ref, dinv_ref, b_ref, o_ref):
    o = dinv_ref[...] * (z_ref[0] + z_ref[1] + y_ref[...]) + b_ref[...]
    m = jnp.max(o, axis=1, keepdims=True)
    t = o - m
    o_ref[...] = t - jnp.log(jnp.sum(jnp.exp(t), axis=1, keepdims=True))


_final = pl.pallas_call(
    _final_body,
    grid=(GRID,),
    in_specs=[
        pl.BlockSpec((NC, RBLK, D), lambda i: (0, i, 0)),
        pl.BlockSpec((RBLK, D), lambda i: (i, 0)),
        pl.BlockSpec((RBLK, D), lambda i: (i, 0)),
        pl.BlockSpec((D,), lambda i: (0,)),
    ],
    out_specs=pl.BlockSpec((RBLK, D), lambda i: (i, 0)),
    out_shape=jax.ShapeDtypeStruct((N, D), jnp.float32),
)


def kernel(x, edge_index, W1, b1, W2, b2):
    idxp = edge_index.astype(jnp.int32)
    ones_rows = jnp.ones((CHUNK, DEGW), jnp.float32)
    zer_d = jnp.zeros((CHUNK, D), jnp.float32)

    degp = _deg_kernel(idxp, ones_rows, zer_d)
    y1, dinvb = _dinv_mm(degp, x, W1)
    z1 = _edge_kernel(y1, idxp, zer_d)
    y2 = _layer2(z1, y1, dinvb, b1, W2)
    z2 = _edge_kernel(y2, idxp, zer_d)
    return _final(z2, y2, dinvb, b2)


# speedup vs baseline: 29.1333x; 1.0127x over previous
import functools

import jax
import jax.numpy as jnp
from jax import lax
from jax.experimental import pallas as pl
from jax.experimental.pallas import tpu as pltpu
from jax.experimental.pallas import tpu_sc as plsc

N = 10000
E = 320000
D = 128
NC = 2
NS = 16
NW = NC * NS
CHUNK = 128
ROWS = E // CHUNK
PAIRS = ROWS // 2
PPW = PAIRS // NW
ZROWS = 10112
ZSTRIPE = ZROWS // NS
DEGW = 128
RBLK = 5000
GRID = N // RBLK

_sc_mesh = plsc.VectorSubcoreMesh(
    core_axis_name="c", subcore_axis_name="s", num_cores=NC, num_subcores=NS
)


def _init_stripe(zer_hbm, zbuf, acc, s):
    pltpu.sync_copy(zer_hbm, zbuf)
    for i in range(ZSTRIPE // CHUNK):
        pltpu.sync_copy(zbuf, acc.at[pl.ds(s * ZSTRIPE + i * CHUNK, CHUNK)])
    rem = ZSTRIPE % CHUNK
    if rem:
        pltpu.sync_copy(
            zbuf.at[pl.ds(0, rem)],
            acc.at[pl.ds(s * ZSTRIPE + ZSTRIPE - rem, rem)],
        )


@functools.partial(
    pl.kernel,
    out_type=jax.ShapeDtypeStruct((NC, ZROWS, DEGW), jnp.float32),
    mesh=_sc_mesh,
    scratch_types=[
        pltpu.VMEM((2, CHUNK), jnp.int32),
        pltpu.VMEM((2, CHUNK), jnp.int32),
        pltpu.VMEM((CHUNK, DEGW), jnp.float32),
        pltpu.VMEM((CHUNK, DEGW), jnp.float32),
        pltpu.VMEM_SHARED((ZROWS, DEGW), jnp.float32),
        pltpu.SemaphoreType.DMA,
    ],
)
def _deg_kernel(idx_hbm, ones_hbm, zer_hbm, out_hbm,
                q0, q1, onesv, zbuf, acc, sem_i):
    c = lax.axis_index("c")
    s = lax.axis_index("s")
    wid = c * NS + s
    _init_stripe(zer_hbm, zbuf, acc, s)
    pltpu.sync_copy(ones_hbm, onesv)
    plsc.subcore_barrier()

    p0 = wid * PPW

    def load(p, q):
        pltpu.async_copy(idx_hbm.at[1, pl.ds(p * 2 * CHUNK, CHUNK)],
                         q.at[0], sem_i)
        pltpu.async_copy(idx_hbm.at[1, pl.ds(p * 2 * CHUNK + CHUNK, CHUNK)],
                         q.at[1], sem_i)

    def drain(p, q):
        pltpu.make_async_copy(idx_hbm.at[1, pl.ds(p * 2 * CHUNK, CHUNK)],
                              q.at[0], sem_i).wait()
        pltpu.make_async_copy(idx_hbm.at[1, pl.ds(p * 2 * CHUNK, CHUNK)],
                              q.at[1], sem_i).wait()

    def scat(q):
        pltpu.sync_copy(onesv, acc.at[q.at[0]], add=True)
        pltpu.sync_copy(onesv, acc.at[q.at[1]], add=True)

    load(p0, q0)
    drain(p0, q0)

    def dbody(k, carry):
        j0 = p0 + 2 * k
        load(j0 + 1, q1)
        scat(q0)
        drain(j0 + 1, q1)
        load(j0 + 2, q0)
        scat(q1)
        drain(j0 + 2, q0)
        return carry

    lax.fori_loop(0, (PPW - 1) // 2, dbody, 0)
    scat(q0)

    @pl.when(wid < 2)
    def _():
        load(NW * PPW + wid, q1)
        drain(NW * PPW + wid, q1)
        scat(q1)

    plsc.subcore_barrier()
    pltpu.sync_copy(
        acc.at[pl.ds(s * ZSTRIPE, ZSTRIPE)],
        out_hbm.at[c, pl.ds(s * ZSTRIPE, ZSTRIPE)],
    )


@functools.partial(
    pl.kernel,
    out_type=jax.ShapeDtypeStruct((NC, ZROWS, D), jnp.float32),
    mesh=_sc_mesh,
    scratch_types=[
        pltpu.VMEM((2, 2, CHUNK), jnp.int32),
        pltpu.VMEM((2, 2, CHUNK), jnp.int32),
        pltpu.VMEM((CHUNK, D), jnp.float32),
        pltpu.VMEM((CHUNK, D), jnp.float32),
        pltpu.VMEM_SHARED((ZROWS, D), jnp.float32),
        pltpu.SemaphoreType.DMA,
        pltpu.SemaphoreType.DMA,
        pltpu.SemaphoreType.DMA,
        pltpu.SemaphoreType.DMA,
    ],
)
def _edge_kernel(y_hbm, idx_hbm, zer_hbm, out_hbm,
                 q0, q1, rows_a, rows_b, acc, sem_a, sem_b, sem_i1, sem_i0):
    c = lax.axis_index("c")
    s = lax.axis_index("s")
    wid = c * NS + s
    _init_stripe(zer_hbm, rows_a, acc, s)
    plsc.subcore_barrier()

    p0 = wid * PPW

    def halfstep(qc, qn, pn, sem_in):
        pltpu.async_copy(y_hbm.at[qc.at[0, 1]], rows_b, sem_b)
        pltpu.async_copy(idx_hbm.at[0, pl.ds(pn * 2 * CHUNK, CHUNK)],
                         qn.at[0, 0], sem_in)
        pltpu.async_copy(idx_hbm.at[0, pl.ds(pn * 2 * CHUNK + CHUNK, CHUNK)],
                         qn.at[0, 1], sem_in)
        pltpu.async_copy(idx_hbm.at[1, pl.ds(pn * 2 * CHUNK, CHUNK)],
                         qn.at[1, 0], sem_in)
        pltpu.async_copy(idx_hbm.at[1, pl.ds(pn * 2 * CHUNK + CHUNK, CHUNK)],
                         qn.at[1, 1], sem_in)
        pltpu.make_async_copy(y_hbm.at[qc.at[0, 0]], rows_a, sem_a).wait()
        pltpu.sync_copy(rows_a, acc.at[qc.at[1, 0]], add=True)
        for _k in range(4):
            pltpu.make_async_copy(idx_hbm.at[0, pl.ds(pn * 2 * CHUNK, CHUNK)],
                                  qn.at[0, 0], sem_in).wait()
        pltpu.async_copy(y_hbm.at[qn.at[0, 0]], rows_a, sem_a)
        pltpu.make_async_copy(y_hbm.at[qc.at[0, 1]], rows_b, sem_b).wait()
        pltpu.sync_copy(rows_b, acc.at[qc.at[1, 1]], add=True)

    pltpu.sync_copy(idx_hbm.at[0, pl.ds(p0 * 2 * CHUNK, CHUNK)], q0.at[0, 0])
    pltpu.sync_copy(idx_hbm.at[0, pl.ds(p0 * 2 * CHUNK + CHUNK, CHUNK)], q0.at[0, 1])
    pltpu.sync_copy(idx_hbm.at[1, pl.ds(p0 * 2 * CHUNK, CHUNK)], q0.at[1, 0])
    pltpu.sync_copy(idx_hbm.at[1, pl.ds(p0 * 2 * CHUNK + CHUNK, CHUNK)], q0.at[1, 1])
    pltpu.async_copy(y_hbm.at[q0.at[0, 0]], rows_a, sem_a)

    def dbody(q, carry):
        j0 = p0 + 2 * q
        halfstep(q0, q1, j0 + 1, sem_i1)
        halfstep(q1, q0, j0 + 2, sem_i0)
        return carry

    lax.fori_loop(0, (PPW - 1) // 2, dbody, 0)

    pltpu.async_copy(y_hbm.at[q0.at[0, 1]], rows_b, sem_b)
    pltpu.make_async_copy(y_hbm.at[q0.at[0, 0]], rows_a, sem_a).wait()
    pltpu.sync_copy(rows_a, acc.at[q0.at[1, 0]], add=True)
    pltpu.make_async_copy(y_hbm.at[q0.at[0, 1]], rows_b, sem_b).wait()
    pltpu.sync_copy(rows_b, acc.at[q0.at[1, 1]], add=True)

    @pl.when(wid < 2)
    def _():
        pe = NW * PPW + wid
        pltpu.sync_copy(idx_hbm.at[0, pl.ds(pe * 2 * CHUNK, CHUNK)], q0.at[0, 0])
        pltpu.sync_copy(idx_hbm.at[0, pl.ds(pe * 2 * CHUNK + CHUNK, CHUNK)], q0.at[0, 1])
        pltpu.sync_copy(idx_hbm.at[1, pl.ds(pe * 2 * CHUNK, CHUNK)], q0.at[1, 0])
        pltpu.sync_copy(idx_hbm.at[1, pl.ds(pe * 2 * CHUNK + CHUNK, CHUNK)], q0.at[1, 1])
        cp_a = pltpu.async_copy(y_hbm.at[q0.at[0, 0]], rows_a, sem_a)
        cp_b = pltpu.async_copy(y_hbm.at[q0.at[0, 1]], rows_b, sem_b)
        cp_a.wait()
        pltpu.sync_copy(rows_a, acc.at[q0.at[1, 0]], add=True)
        cp_b.wait()
        pltpu.sync_copy(rows_b, acc.at[q0.at[1, 1]], add=True)

    plsc.subcore_barrier()
    pltpu.sync_copy(
        acc.at[pl.ds(s * ZSTRIPE, ZSTRIPE)],
        out_hbm.at[c, pl.ds(s * ZSTRIPE, ZSTRIPE)],
    )


def _dinv_mm_body(dg_ref, x_ref, w_ref, y_ref, dinv_ref):
    dg = dg_ref[...]
    d = dg[0, :, 0:1] + dg[1, :, 0:1] + 1.0
    dinvb = jnp.broadcast_to(lax.rsqrt(d), (RBLK, D))
    xw = jnp.dot(x_ref[...], w_ref[...], preferred_element_type=jnp.float32)
    y_ref[...] = xw * dinvb
    dinv_ref[...] = dinvb


_dinv_mm = pl.pallas_call(
    _dinv_mm_body,
    grid=(GRID,),
    in_specs=[
        pl.BlockSpec((NC, RBLK, DEGW), lambda i: (0, i, 0)),
        pl.BlockSpec((RBLK, D), lambda i: (i, 0)),
        pl.BlockSpec((D, D), lambda i: (0, 0)),
    ],
    out_specs=[
        pl.BlockSpec((RBLK, D), lambda i: (i, 0)),
        pl.BlockSpec((RBLK, D), lambda i: (i, 0)),
    ],
    out_shape=[
        jax.ShapeDtypeStruct((N, D), jnp.float32),
        jax.ShapeDtypeStruct((N, D), jnp.float32),
    ],
)


def _layer2_body(z_ref, y_ref, dinv_ref, b_ref, w_ref, o_ref):
    zsum = z_ref[0] + z_ref[1]
    h = jnp.maximum(dinv_ref[...] * (zsum + y_ref[...]) + b_ref[...], 0.0)
    hw = jnp.dot(h, w_ref[...], preferred_element_type=jnp.float32)
    o_ref[...] = hw * dinv_ref[...]


_layer2 = pl.pallas_call(
    _layer2_body,
    grid=(GRID,),
    in_specs=[
        pl.BlockSpec((NC, RBLK, D), lambda i: (0, i, 0)),
        pl.BlockSpec((RBLK, D), lambda i: (i, 0)),
        pl.BlockSpec((RBLK, D), lambda i: (i, 0)),
        pl.BlockSpec((D,), lambda i: (0,)),
        pl.BlockSpec((D, D), lambda i: (0, 0)),
    ],
    out_specs=pl.BlockSpec((RBLK, D), lambda i: (i, 0)),
    out_shape=jax.ShapeDtypeStruct((N, D), jnp.float32),
)


def _final_body(z_ref, y_ref, dinv_ref, b_ref, o_ref):
    o = dinv_ref[...] * (z_ref[0] + z_ref[1] + y_ref[...]) + b_ref[...]
    m = jnp.max(o, axis=1, keepdims=True)
    t = o - m
    o_ref[...] = t - jnp.log(jnp.sum(jnp.exp(t), axis=1, keepdims=True))


_final = pl.pallas_call(
    _final_body,
    grid=(GRID,),
    in_specs=[
        pl.BlockSpec((NC, RBLK, D), lambda i: (0, i, 0)),
        pl.BlockSpec((RBLK, D), lambda i: (i, 0)),
        pl.BlockSpec((RBLK, D), lambda i: (i, 0)),
        pl.BlockSpec((D,), lambda i: (0,)),
    ],
    out_specs=pl.BlockSpec((RBLK, D), lambda i: (i, 0)),
    out_shape=jax.ShapeDtypeStruct((N, D), jnp.float32),
)


def kernel(x, edge_index, W1, b1, W2, b2):
    idxp = edge_index.astype(jnp.int32)
    ones_rows = jnp.ones((CHUNK, DEGW), jnp.float32)
    zer_d = jnp.zeros((CHUNK, D), jnp.float32)

    degp = _deg_kernel(idxp, ones_rows, zer_d)
    y1, dinvb = _dinv_mm(degp, x, W1)
    z1 = _edge_kernel(y1, idxp, zer_d)
    y2 = _layer2(z1, y1, dinvb, b1, W2)
    z2 = _edge_kernel(y2, idxp, zer_d)
    return _final(z2, y2, dinvb, b2)
